# baseline (speedup 1.0000x reference)
import functools

import jax
import jax.numpy as jnp
from jax import lax
from jax.experimental import pallas as pl
from jax.experimental.pallas import tpu as pltpu
from jax.experimental.pallas import tpu_sc as plsc

_TIMESTEPS = 1000
_ZERO_WEIGHTING = 1.0
_NONZERO_WEIGHTING = 1.0

_B, _C, _H, _W = 4, 1, 96, 96
_N = _B * _C * _H * _W
_N_NONZERO = float(_N)
_N_ZERO = float(_N * _TIMESTEPS - _N)
_SCALE = _ZERO_WEIGHTING / _N_ZERO + _NONZERO_WEIGHTING / _N_NONZERO

_NUM_CORES = 1
_NUM_TILES = 16
_NUM_WORKERS = _NUM_CORES * _NUM_TILES
_CHUNK = _N // _NUM_WORKERS
_LANES = 16
_STEPS = _CHUNK // _LANES
_UNROLL = 4


def _bin_index(x):
    y = x * jnp.float32(_TIMESTEPS) - jnp.float32(1.0)
    return jnp.maximum(y, jnp.float32(0.0)).astype(jnp.int32)


def _sc_count_kernel(r_hbm, t_hbm, rows_hbm, r_buf, t_buf, acc_buf,
                     sem1, sem2):
    wid = lax.axis_index("s") * _NUM_CORES + lax.axis_index("c")
    base = wid * _CHUNK
    half = _CHUNK // 2

    c1r = pltpu.async_copy(r_hbm.at[pl.ds(base, half)],
                           r_buf.at[pl.ds(0, half)], sem1)
    c1t = pltpu.async_copy(t_hbm.at[pl.ds(base, half)],
                           t_buf.at[pl.ds(0, half)], sem1)
    c2r = pltpu.async_copy(r_hbm.at[pl.ds(base + half, half)],
                           r_buf.at[pl.ds(half, half)], sem2)
    c2t = pltpu.async_copy(t_hbm.at[pl.ds(base + half, half)],
                           t_buf.at[pl.ds(half, half)], sem2)

    def count(start, acc):
        def body(i, acc):
            for u in range(_UNROLL):
                off = start + (i * _UNROLL + u) * _LANES
                r = r_buf[pl.ds(off, _LANES)]
                t = t_buf[pl.ds(off, _LANES)]
                ne = _bin_index(r) != _bin_index(t)
                acc = acc + jnp.where(ne, jnp.int32(1), jnp.int32(0))
            return acc
        return lax.fori_loop(0, (_STEPS // 2) // _UNROLL, body, acc)

    c1r.wait()
    c1t.wait()
    acc = count(0, jnp.zeros((_LANES,), jnp.int32))
    c2r.wait()
    c2t.wait()
    acc = count(half, acc)
    acc_buf[0, :] = acc
    pltpu.sync_copy(acc_buf, rows_hbm.at[pl.ds(wid, 1)])


def _tc_finish_kernel(rows_ref, out_ref):
    total = jnp.sum(rows_ref[...].astype(jnp.float32))
    out_ref[0, 0] = total * jnp.float32(_SCALE)


@jax.jit
def _loss(r_flat, t_flat):
    mesh = plsc.VectorSubcoreMesh(
        core_axis_name="c", subcore_axis_name="s", num_cores=_NUM_CORES
    )
    count = functools.partial(
        pl.kernel,
        mesh=mesh,
        out_type=jax.ShapeDtypeStruct((_NUM_WORKERS, _LANES), jnp.int32),
        scratch_types=[
            pltpu.VMEM((_CHUNK,), jnp.float32),
            pltpu.VMEM((_CHUNK,), jnp.float32),
            pltpu.VMEM((1, _LANES), jnp.int32),
            pltpu.SemaphoreType.DMA,
            pltpu.SemaphoreType.DMA,
        ],
    )(_sc_count_kernel)
    rows = count(r_flat, t_flat)
    return jnp.sum(rows.astype(jnp.float32)) * jnp.float32(_SCALE)


def kernel(reconstructed_image, target_image):
    r_flat = reconstructed_image.reshape(_N)
    t_flat = target_image.reshape(_N)
    return _loss(r_flat, t_flat)


# speedup vs baseline: 1.0524x; 1.0524x over previous
import functools

import jax
import jax.numpy as jnp
from jax import lax
from jax.experimental import pallas as pl
from jax.experimental.pallas import tpu as pltpu
from jax.experimental.pallas import tpu_sc as plsc

_TIMESTEPS = 1000
_ZERO_WEIGHTING = 1.0
_NONZERO_WEIGHTING = 1.0

_B, _C, _H, _W = 4, 1, 96, 96
_N = _B * _C * _H * _W
_N_NONZERO = float(_N)
_N_ZERO = float(_N * _TIMESTEPS - _N)
_SCALE = _ZERO_WEIGHTING / _N_ZERO + _NONZERO_WEIGHTING / _N_NONZERO

_NUM_CORES = 1
_NUM_TILES = 16
_NUM_WORKERS = _NUM_CORES * _NUM_TILES
_CHUNK = _N // _NUM_WORKERS
_LANES = 16
_STEPS = _CHUNK // _LANES
_UNROLL = 4


def _bin_index(x):
    y = x * jnp.float32(_TIMESTEPS) - jnp.float32(1.0)
    return jnp.maximum(y, jnp.float32(0.0)).astype(jnp.int32)


def _sc_count_kernel(r_hbm, t_hbm, rows_hbm, r_buf, t_buf, acc_buf,
                     sem1, sem2):
    wid = lax.axis_index("s") * _NUM_CORES + lax.axis_index("c")
    base = wid * _CHUNK
    half = _CHUNK // 2

    c1r = pltpu.async_copy(r_hbm.at[pl.ds(base, half)],
                           r_buf.at[pl.ds(0, half)], sem1)
    c1t = pltpu.async_copy(t_hbm.at[pl.ds(base, half)],
                           t_buf.at[pl.ds(0, half)], sem1)
    c2r = pltpu.async_copy(r_hbm.at[pl.ds(base + half, half)],
                           r_buf.at[pl.ds(half, half)], sem2)
    c2t = pltpu.async_copy(t_hbm.at[pl.ds(base + half, half)],
                           t_buf.at[pl.ds(half, half)], sem2)

    def count(start, acc):
        def body(i, acc):
            for u in range(_UNROLL):
                off = start + (i * _UNROLL + u) * _LANES
                r = r_buf[pl.ds(off, _LANES)]
                t = t_buf[pl.ds(off, _LANES)]
                ne = _bin_index(r) != _bin_index(t)
                acc = acc + jnp.where(ne, jnp.int32(1), jnp.int32(0))
            return acc
        return lax.fori_loop(0, (_STEPS // 2) // _UNROLL, body, acc)

    c1r.wait()
    c1t.wait()
    acc = count(0, jnp.zeros((_LANES,), jnp.int32))
    c2r.wait()
    c2t.wait()
    acc = count(half, acc)
    acc_buf[0, :] = acc
    pltpu.sync_copy(acc_buf, rows_hbm.at[pl.ds(wid, 1)])


def _tc_finish_kernel(rows_ref, out_ref):
    total = jnp.sum(rows_ref[...].astype(jnp.float32))
    out_ref[0, 0] = total * jnp.float32(_SCALE)


@jax.jit
def _loss(r_flat, t_flat):
    mesh = plsc.VectorSubcoreMesh(
        core_axis_name="c", subcore_axis_name="s", num_cores=_NUM_CORES
    )
    count = functools.partial(
        pl.kernel,
        mesh=mesh,
        out_type=jax.ShapeDtypeStruct((_NUM_WORKERS, _LANES), jnp.int32),
        scratch_types=[
            pltpu.VMEM((_CHUNK,), jnp.float32),
            pltpu.VMEM((_CHUNK,), jnp.float32),
            pltpu.VMEM((1, _LANES), jnp.int32),
            pltpu.SemaphoreType.DMA,
            pltpu.SemaphoreType.DMA,
        ],
    )(_sc_count_kernel)
    rows = count(r_flat, t_flat)
    loss = pl.pallas_call(
        _tc_finish_kernel,
        out_shape=jax.ShapeDtypeStruct((1, 1), jnp.float32),
        out_specs=pl.BlockSpec(memory_space=pltpu.SMEM),
    )(rows)
    return loss[0, 0]


def kernel(reconstructed_image, target_image):
    r_flat = reconstructed_image.reshape(_N)
    t_flat = target_image.reshape(_N)
    return _loss(r_flat, t_flat)


# speedup vs baseline: 1.0551x; 1.0026x over previous
import functools

import jax
import jax.numpy as jnp
from jax import lax
from jax.experimental import pallas as pl
from jax.experimental.pallas import tpu as pltpu
from jax.experimental.pallas import tpu_sc as plsc

_TIMESTEPS = 1000
_ZERO_WEIGHTING = 1.0
_NONZERO_WEIGHTING = 1.0

_B, _C, _H, _W = 4, 1, 96, 96
_N = _B * _C * _H * _W
_N_NONZERO = float(_N)
_N_ZERO = float(_N * _TIMESTEPS - _N)
_SCALE = _ZERO_WEIGHTING / _N_ZERO + _NONZERO_WEIGHTING / _N_NONZERO

_NUM_CORES = 1
_NUM_TILES = 16
_NUM_WORKERS = _NUM_CORES * _NUM_TILES
_CHUNK = _N // _NUM_WORKERS
_LANES = 16
_STEPS = _CHUNK // _LANES
_UNROLL = 4


def _bin_index(x):
    y = x * jnp.float32(_TIMESTEPS) - jnp.float32(1.0)
    return jnp.maximum(y, jnp.float32(0.0)).astype(jnp.int32)


def _sc_count_kernel(r_hbm, t_hbm, rows_hbm, r_buf, t_buf, acc_buf,
                     sem1, sem2):
    wid = lax.axis_index("s") * _NUM_CORES + lax.axis_index("c")
    base = wid * _CHUNK
    h1 = _CHUNK // 4
    h2 = _CHUNK - h1

    c1r = pltpu.async_copy(r_hbm.at[pl.ds(base, h1)],
                           r_buf.at[pl.ds(0, h1)], sem1)
    c1t = pltpu.async_copy(t_hbm.at[pl.ds(base, h1)],
                           t_buf.at[pl.ds(0, h1)], sem1)
    c2r = pltpu.async_copy(r_hbm.at[pl.ds(base + h1, h2)],
                           r_buf.at[pl.ds(h1, h2)], sem2)
    c2t = pltpu.async_copy(t_hbm.at[pl.ds(base + h1, h2)],
                           t_buf.at[pl.ds(h1, h2)], sem2)

    def count(start, nsteps, acc):
        def body(i, acc):
            for u in range(_UNROLL):
                off = start + (i * _UNROLL + u) * _LANES
                r = r_buf[pl.ds(off, _LANES)]
                t = t_buf[pl.ds(off, _LANES)]
                ne = _bin_index(r) != _bin_index(t)
                acc = acc + jnp.where(ne, jnp.int32(1), jnp.int32(0))
            return acc
        return lax.fori_loop(0, nsteps // _UNROLL, body, acc)

    c1r.wait()
    c1t.wait()
    acc = count(0, h1 // _LANES, jnp.zeros((_LANES,), jnp.int32))
    c2r.wait()
    c2t.wait()
    acc = count(h1, h2 // _LANES, acc)
    acc_buf[0, :] = acc
    pltpu.sync_copy(acc_buf, rows_hbm.at[pl.ds(wid, 1)])


def _tc_finish_kernel(rows_ref, out_ref):
    total = jnp.sum(rows_ref[...].astype(jnp.float32))
    out_ref[0, 0] = total * jnp.float32(_SCALE)


@jax.jit
def _loss(r_flat, t_flat):
    mesh = plsc.VectorSubcoreMesh(
        core_axis_name="c", subcore_axis_name="s", num_cores=_NUM_CORES
    )
    count = functools.partial(
        pl.kernel,
        mesh=mesh,
        out_type=jax.ShapeDtypeStruct((_NUM_WORKERS, _LANES), jnp.int32),
        scratch_types=[
            pltpu.VMEM((_CHUNK,), jnp.float32),
            pltpu.VMEM((_CHUNK,), jnp.float32),
            pltpu.VMEM((1, _LANES), jnp.int32),
            pltpu.SemaphoreType.DMA,
            pltpu.SemaphoreType.DMA,
        ],
    )(_sc_count_kernel)
    rows = count(r_flat, t_flat)
    loss = pl.pallas_call(
        _tc_finish_kernel,
        out_shape=jax.ShapeDtypeStruct((1, 1), jnp.float32),
        out_specs=pl.BlockSpec(memory_space=pltpu.SMEM),
    )(rows)
    return loss[0, 0]


def kernel(reconstructed_image, target_image):
    r_flat = reconstructed_image.reshape(_N)
    t_flat = target_image.reshape(_N)
    return _loss(r_flat, t_flat)
